# baseline (speedup 1.0000x reference)
import functools

import jax
import jax.numpy as jnp
from jax import lax
from jax.experimental import pallas as pl
from jax.experimental.pallas import tpu as pltpu
from jax.experimental.pallas import tpu_sc as plsc

_NUM_CORES = 2
_NUM_SUBCORES = 16
_NUM_WORKERS = _NUM_CORES * _NUM_SUBCORES

_CHUNK = 400
_NBUF = 4
_INFLIGHT = 3


def _gather_body(ids_hbm, table_hbm, out_hbm, idx_v, rows_v, sems_i, sems_g, sems_o):
    n_ids = ids_hbm.shape[0]
    b_per_w = n_ids // _NUM_WORKERS
    wid = lax.axis_index("s") * _NUM_CORES + lax.axis_index("c")
    base = wid * b_per_w
    nchunks = b_per_w // _CHUNK

    def idx_start(i, b):
        pltpu.async_copy(ids_hbm.at[pl.ds(base + i * _CHUNK, _CHUNK)],
                         idx_v.at[b], sems_i[b])

    def idx_wait(b):
        pltpu.make_async_copy(ids_hbm.at[pl.ds(base, _CHUNK)],
                              idx_v.at[b], sems_i[b]).wait()

    def gather_start(b):
        pltpu.async_copy(table_hbm.at[pl.ds(0, 16)], rows_v.at[b, pl.ds(0, 16)], sems_g[b])

    def gather_wait(b):
        pltpu.make_async_copy(table_hbm.at[pl.ds(0, 16)],
                              rows_v.at[b, pl.ds(0, 16)], sems_g[b]).wait()

    def out_start(i, b):
        pltpu.async_copy(rows_v.at[b],
                         out_hbm.at[pl.ds(base + i * _CHUNK, _CHUNK)], sems_o[b])

    def out_wait(b):
        pltpu.make_async_copy(rows_v.at[b],
                              out_hbm.at[pl.ds(base, _CHUNK)], sems_o[b]).wait()

    for k in range(_NBUF):
        idx_start(k, k)
    for k in range(_INFLIGHT):
        idx_wait(k)
        gather_start(k)

    def group(g, carry):
        for b in range(_NBUF):
            i = g * _NBUF + b
            bg = (b + _INFLIGHT) % _NBUF

            @pl.when(i + _INFLIGHT < nchunks)
            def _():
                idx_wait(bg)

                @pl.when(i >= _NBUF - _INFLIGHT)
                def _():
                    out_wait(bg)

                gather_start(bg)

            gather_wait(b)
            out_start(i, b)

            @pl.when(i + _NBUF < nchunks)
            def _():
                idx_start(i + _NBUF, b)
        return carry

    lax.fori_loop(0, nchunks // _NBUF, group, 0)

    for b in range(_NBUF):
        out_wait(b)


@jax.jit
def kernel(token_ids, weight):
    orig_shape = token_ids.shape
    flat_ids = token_ids.reshape(-1).astype(jnp.int32)
    n = flat_ids.shape[0]
    dim = weight.shape[1]

    mesh = plsc.VectorSubcoreMesh(
        core_axis_name="c",
        subcore_axis_name="s",
        num_cores=_NUM_CORES,
        num_subcores=_NUM_SUBCORES,
    )
    dma_sems = tuple(pltpu.SemaphoreType.DMA for _ in range(_NBUF))
    run = pl.kernel(
        _gather_body,
        out_type=jax.ShapeDtypeStruct((n, dim), weight.dtype),
        mesh=mesh,
        scratch_types=[
            pltpu.VMEM((_NBUF, _CHUNK), jnp.int32),
            pltpu.VMEM((_NBUF, _CHUNK, dim), weight.dtype),
            dma_sems,
            dma_sems,
            dma_sems,
        ],
        compiler_params=pltpu.CompilerParams(use_tc_tiling_on_sc=False),
    )
    out = run(flat_ids, weight)
    return out.reshape(*orig_shape, dim)


# speedup vs baseline: 1.0876x; 1.0876x over previous
import functools

import jax
import jax.numpy as jnp
from jax import lax
from jax.experimental import pallas as pl
from jax.experimental.pallas import tpu as pltpu
from jax.experimental.pallas import tpu_sc as plsc

_NUM_CORES = 2
_NUM_SUBCORES = 16
_NUM_WORKERS = _NUM_CORES * _NUM_SUBCORES

_CHUNK = 256
_NBUF = 2
_WIDE = 8
_WCHUNK = _CHUNK // _WIDE
_LANES = 16


def _body(ids_hbm, table_hbm, out_hbm, idx_v, rows_v, wide_v, widx_v,
          sems_i, sems_g, sems_o):
    n_ids = ids_hbm.shape[0]
    dim = table_hbm.shape[1]
    b_per_w = n_ids // _NUM_WORKERS
    wid = lax.axis_index("s") * _NUM_CORES + lax.axis_index("c")
    base = wid * b_per_w
    wbase = base // _WIDE
    nchunks = b_per_w // _CHUNK

    def idx_start(i, b):
        pltpu.async_copy(ids_hbm.at[pl.ds(base + i * _CHUNK, _CHUNK)],
                         idx_v.at[b], sems_i[b])

    def idx_wait(b):
        pltpu.make_async_copy(ids_hbm.at[pl.ds(base, _CHUNK)],
                              idx_v.at[b], sems_i[b]).wait()

    def gather_start(b):
        pltpu.async_copy(table_hbm.at[idx_v.at[b]], rows_v.at[b], sems_g[b])

    def gather_wait(b):
        pltpu.make_async_copy(table_hbm.at[idx_v.at[b]],
                              rows_v.at[b], sems_g[b]).wait()

    def compact(i, b):
        iota = lax.iota(jnp.int32, _LANES)

        def pack(g, carry):
            for r8 in range(_WIDE):
                for c in range(dim // _LANES):
                    vec = rows_v[b, g * _WIDE + r8, pl.ds(c * _LANES, _LANES)]
                    wide_v[b, g, pl.ds(r8 * dim + c * _LANES, _LANES)] = vec
            return carry

        lax.fori_loop(0, _WCHUNK, pack, 0)
        first = wbase + i * _WCHUNK
        for s in range(_WCHUNK // _LANES):
            widx_v[b, pl.ds(s * _LANES, _LANES)] = iota + (first + s * _LANES)

    def out_start(b):
        pltpu.async_copy(wide_v.at[b], out_hbm.at[widx_v.at[b]], sems_o[b])

    def out_wait(b):
        pltpu.make_async_copy(wide_v.at[b], out_hbm.at[widx_v.at[b]],
                              sems_o[b]).wait()

    for k in range(_NBUF):
        idx_start(k, k)
    idx_wait(0)
    gather_start(0)

    def group(g, carry):
        for b in range(_NBUF):
            i = g * _NBUF + b
            o = (b + 1) % _NBUF

            @pl.when(i + 1 < nchunks)
            def _():
                idx_wait(o)

                @pl.when(i >= 1)
                def _():
                    out_wait(o)

                gather_start(o)

            gather_wait(b)
            compact(i, b)
            out_start(b)

            @pl.when(i + _NBUF < nchunks)
            def _():
                idx_start(i + _NBUF, b)
        return carry

    lax.fori_loop(0, nchunks // _NBUF, group, 0)

    for b in range(_NBUF):
        out_wait(b)


@jax.jit
def kernel(token_ids, weight):
    orig_shape = token_ids.shape
    flat_ids = token_ids.reshape(-1).astype(jnp.int32)
    n = flat_ids.shape[0]
    dim = weight.shape[1]

    mesh = plsc.VectorSubcoreMesh(
        core_axis_name="c",
        subcore_axis_name="s",
        num_cores=_NUM_CORES,
        num_subcores=_NUM_SUBCORES,
    )
    dma_sems = tuple(pltpu.SemaphoreType.DMA for _ in range(_NBUF))
    run = pl.kernel(
        _body,
        out_type=jax.ShapeDtypeStruct((n // _WIDE, _WIDE * dim), weight.dtype),
        mesh=mesh,
        scratch_types=[
            pltpu.VMEM((_NBUF, _CHUNK), jnp.int32),
            pltpu.VMEM((_NBUF, _CHUNK, dim), weight.dtype),
            pltpu.VMEM((_NBUF, _WCHUNK, _WIDE * dim), weight.dtype),
            pltpu.VMEM((_NBUF, _WCHUNK), jnp.int32),
            dma_sems,
            dma_sems,
            dma_sems,
        ],
        compiler_params=pltpu.CompilerParams(use_tc_tiling_on_sc=False),
    )
    out = run(flat_ids, weight)
    return out.reshape(*orig_shape, dim)


# speedup vs baseline: 1.0982x; 1.0097x over previous
import functools

import jax
import jax.numpy as jnp
from jax import lax
from jax.experimental import pallas as pl
from jax.experimental.pallas import tpu as pltpu
from jax.experimental.pallas import tpu_sc as plsc

_NUM_CORES = 2
_NUM_SUBCORES = 16
_NUM_WORKERS = _NUM_CORES * _NUM_SUBCORES

_CHUNK = 400
_NBUF = 4
_INFLIGHT = 3


def _gather_body(ids_hbm, table_hbm, out_hbm, idx_v, rows_v, sems_i, sems_g, sems_o):
    n_ids = ids_hbm.shape[0]
    b_per_w = n_ids // _NUM_WORKERS
    wid = lax.axis_index("s") * _NUM_CORES + lax.axis_index("c")
    base = wid * b_per_w
    nchunks = b_per_w // _CHUNK

    def idx_start(i, b):
        pltpu.async_copy(ids_hbm.at[pl.ds(base + i * _CHUNK, _CHUNK)],
                         idx_v.at[b], sems_i[b])

    def idx_wait(b):
        pltpu.make_async_copy(ids_hbm.at[pl.ds(base, _CHUNK)],
                              idx_v.at[b], sems_i[b]).wait()

    def gather_start(b):
        pltpu.async_copy(table_hbm.at[idx_v.at[b]], rows_v.at[b], sems_g[b])

    def gather_wait(b):
        pltpu.make_async_copy(table_hbm.at[idx_v.at[b]],
                              rows_v.at[b], sems_g[b]).wait()

    def out_start(i, b):
        pltpu.async_copy(rows_v.at[b],
                         out_hbm.at[pl.ds(base + i * _CHUNK, _CHUNK)], sems_o[b])

    def out_wait(b):
        pltpu.make_async_copy(rows_v.at[b],
                              out_hbm.at[pl.ds(base, _CHUNK)], sems_o[b]).wait()

    for k in range(_NBUF):
        idx_start(k, k)
    for k in range(_INFLIGHT):
        idx_wait(k)
        gather_start(k)

    def group(g, carry):
        for b in range(_NBUF):
            i = g * _NBUF + b
            bg = (b + _INFLIGHT) % _NBUF

            @pl.when(i + _INFLIGHT < nchunks)
            def _():
                idx_wait(bg)

                @pl.when(i >= _NBUF - _INFLIGHT)
                def _():
                    out_wait(bg)

                gather_start(bg)

            gather_wait(b)
            out_start(i, b)

            @pl.when(i + _NBUF < nchunks)
            def _():
                idx_start(i + _NBUF, b)
        return carry

    lax.fori_loop(0, nchunks // _NBUF, group, 0)

    for b in range(_NBUF):
        out_wait(b)


@jax.jit
def kernel(token_ids, weight):
    orig_shape = token_ids.shape
    flat_ids = token_ids.reshape(-1).astype(jnp.int32)
    n = flat_ids.shape[0]
    dim = weight.shape[1]

    mesh = plsc.VectorSubcoreMesh(
        core_axis_name="c",
        subcore_axis_name="s",
        num_cores=_NUM_CORES,
        num_subcores=_NUM_SUBCORES,
    )
    dma_sems = tuple(pltpu.SemaphoreType.DMA for _ in range(_NBUF))
    run = pl.kernel(
        _gather_body,
        out_type=jax.ShapeDtypeStruct((n, dim), weight.dtype),
        mesh=mesh,
        scratch_types=[
            pltpu.VMEM((_NBUF, _CHUNK), jnp.int32),
            pltpu.VMEM((_NBUF, _CHUNK, dim), weight.dtype),
            dma_sems,
            dma_sems,
            dma_sems,
        ],
        compiler_params=pltpu.CompilerParams(use_tc_tiling_on_sc=False),
    )
    out = run(flat_ids, weight)
    return out.reshape(*orig_shape, dim)


# speedup vs baseline: 1.0989x; 1.0006x over previous
import functools

import jax
import jax.numpy as jnp
from jax import lax
from jax.experimental import pallas as pl
from jax.experimental.pallas import tpu as pltpu
from jax.experimental.pallas import tpu_sc as plsc

_NUM_CORES = 2
_NUM_SUBCORES = 16
_NUM_WORKERS = _NUM_CORES * _NUM_SUBCORES

_CHUNK = 320
_NBUF = 5
_INFLIGHT = 4


def _gather_body(ids_hbm, table_hbm, out_hbm, idx_v, rows_v, sems_i, sems_g, sems_o):
    n_ids = ids_hbm.shape[0]
    b_per_w = n_ids // _NUM_WORKERS
    wid = lax.axis_index("s") * _NUM_CORES + lax.axis_index("c")
    base = wid * b_per_w
    nchunks = b_per_w // _CHUNK

    def idx_start(i, b):
        pltpu.async_copy(ids_hbm.at[pl.ds(base + i * _CHUNK, _CHUNK)],
                         idx_v.at[b], sems_i[b])

    def idx_wait(b):
        pltpu.make_async_copy(ids_hbm.at[pl.ds(base, _CHUNK)],
                              idx_v.at[b], sems_i[b]).wait()

    def gather_start(b):
        pltpu.async_copy(table_hbm.at[idx_v.at[b]], rows_v.at[b], sems_g[b])

    def gather_wait(b):
        pltpu.make_async_copy(table_hbm.at[idx_v.at[b]],
                              rows_v.at[b], sems_g[b]).wait()

    def out_start(i, b):
        pltpu.async_copy(rows_v.at[b],
                         out_hbm.at[pl.ds(base + i * _CHUNK, _CHUNK)], sems_o[b])

    def out_wait(b):
        pltpu.make_async_copy(rows_v.at[b],
                              out_hbm.at[pl.ds(base, _CHUNK)], sems_o[b]).wait()

    for k in range(_NBUF):
        idx_start(k, k)
    for k in range(_INFLIGHT):
        idx_wait(k)
        gather_start(k)

    def group(g, carry):
        for b in range(_NBUF):
            i = g * _NBUF + b
            bg = (b + _INFLIGHT) % _NBUF

            @pl.when(i + _INFLIGHT < nchunks)
            def _():
                idx_wait(bg)

                @pl.when(i >= _NBUF - _INFLIGHT)
                def _():
                    out_wait(bg)

                gather_start(bg)

            gather_wait(b)
            out_start(i, b)

            @pl.when(i + _NBUF < nchunks)
            def _():
                idx_start(i + _NBUF, b)
        return carry

    lax.fori_loop(0, nchunks // _NBUF, group, 0)

    for b in range(_NBUF):
        out_wait(b)


@jax.jit
def kernel(token_ids, weight):
    orig_shape = token_ids.shape
    flat_ids = token_ids.reshape(-1).astype(jnp.int32)
    n = flat_ids.shape[0]
    dim = weight.shape[1]

    mesh = plsc.VectorSubcoreMesh(
        core_axis_name="c",
        subcore_axis_name="s",
        num_cores=_NUM_CORES,
        num_subcores=_NUM_SUBCORES,
    )
    dma_sems = tuple(pltpu.SemaphoreType.DMA for _ in range(_NBUF))
    run = pl.kernel(
        _gather_body,
        out_type=jax.ShapeDtypeStruct((n, dim), weight.dtype),
        mesh=mesh,
        scratch_types=[
            pltpu.VMEM((_NBUF, _CHUNK), jnp.int32),
            pltpu.VMEM((_NBUF, _CHUNK, dim), weight.dtype),
            dma_sems,
            dma_sems,
            dma_sems,
        ],
        compiler_params=pltpu.CompilerParams(use_tc_tiling_on_sc=False),
    )
    out = run(flat_ids, weight)
    return out.reshape(*orig_shape, dim)
